# baseline (speedup 1.0000x reference)
import functools

import jax
import jax.numpy as jnp
from jax import lax
from jax.experimental import pallas as pl
from jax.experimental.pallas import tpu as pltpu
from jax.experimental.pallas import tpu_sc as plsc

_NC = 2
_NS = 16
_NW = _NC * _NS
_K = 128
_BLK = 128


def _sc_agg(m, src_r, dst_r, zeros, n_agg, n_pad, d, chunks):
    mesh = plsc.VectorSubcoreMesh(core_axis_name="c", subcore_axis_name="s")

    @functools.partial(
        pl.kernel,
        mesh=mesh,
        out_type=jax.ShapeDtypeStruct((_NC, n_pad, d), jnp.float32),
        scratch_types=[
            pltpu.VMEM((chunks * _K,), jnp.int32),
            pltpu.VMEM((chunks, _K), jnp.int32),
            pltpu.VMEM((_K, d), jnp.float32),
            pltpu.VMEM_SHARED((n_agg, d), jnp.float32),
            pltpu.SemaphoreType.DMA,
        ],
    )
    def body(m_hbm, src_hbm, dst_hbm, z_hbm, out_hbm, src_v, dst_v,
             rows_v, agg_sh, g0):
        c = lax.axis_index("c")
        s = lax.axis_index("s")
        wid = s * _NC + c

        @pl.when(s == 0)
        def _():
            pltpu.sync_copy(z_hbm, agg_sh)

        pltpu.sync_copy(src_hbm.at[wid], src_v)
        pltpu.sync_copy(dst_hbm.at[wid], dst_v)
        plsc.subcore_barrier()

        def step(j, carry):
            pltpu.async_copy(
                m_hbm.at[src_v.at[pl.ds(j * _K, _K)]], rows_v, g0)
            pltpu.make_async_copy(z_hbm.at[pl.ds(0, _K)], rows_v, g0).wait()
            pltpu.sync_copy(rows_v, agg_sh.at[dst_v.at[j]], add=True)
            return carry

        lax.fori_loop(0, chunks, step, 0)
        plsc.subcore_barrier()

        @pl.when(s == 0)
        def _():
            pltpu.sync_copy(agg_sh, out_hbm.at[c].at[pl.ds(0, n_agg)])

    return body(m, src_r, dst_r, zeros)


def _mm_body(h_ref, w_ref, o_ref):
    o_ref[...] = jnp.dot(h_ref[...], w_ref[...],
                         preferred_element_type=jnp.float32)


def _mm(h, w, n_pad, d):
    grid = n_pad // _BLK
    return pl.pallas_call(
        _mm_body,
        grid=(grid,),
        in_specs=[
            pl.BlockSpec((_BLK, d), lambda i: (i, 0)),
            pl.BlockSpec((d, d), lambda i: (0, 0)),
        ],
        out_specs=pl.BlockSpec((_BLK, d), lambda i: (i, 0)),
        out_shape=jax.ShapeDtypeStruct((n_pad, d), jnp.float32),
    )(h, w)


def _gru_body(n_nodes, d, parts_ref, h_ref, wih_ref, whh_ref, bih_ref,
              bhh_ref, wnext_ref, h_out_ref, m_out_ref):
    i = pl.program_id(0)
    agg = parts_ref[0] + parts_ref[1]
    h = h_ref[...]
    cdims = (((1,), (1,)), ((), ()))
    gi = lax.dot_general(agg, wih_ref[...], cdims,
                         preferred_element_type=jnp.float32) + bih_ref[...]
    gh = lax.dot_general(h, whh_ref[...], cdims,
                         preferred_element_type=jnp.float32) + bhh_ref[...]
    r = jax.nn.sigmoid(gi[:, :d] + gh[:, :d])
    z = jax.nn.sigmoid(gi[:, d:2 * d] + gh[:, d:2 * d])
    n = jnp.tanh(gi[:, 2 * d:] + r * gh[:, 2 * d:])
    hn = (1.0 - z) * n + z * h
    row = i * _BLK + lax.broadcasted_iota(jnp.int32, (_BLK, 1), 0)
    hn = jnp.where(row < n_nodes, hn, 0.0)
    h_out_ref[...] = hn
    m_out_ref[...] = jnp.dot(hn, wnext_ref[...],
                             preferred_element_type=jnp.float32)


def _gru(parts, h, w_ih, w_hh, b_ih, b_hh, w_next, n_nodes, n_pad, d):
    grid = n_pad // _BLK
    return pl.pallas_call(
        functools.partial(_gru_body, n_nodes, d),
        grid=(grid,),
        in_specs=[
            pl.BlockSpec((2, _BLK, d), lambda i: (0, i, 0)),
            pl.BlockSpec((_BLK, d), lambda i: (i, 0)),
            pl.BlockSpec((3 * d, d), lambda i: (0, 0)),
            pl.BlockSpec((3 * d, d), lambda i: (0, 0)),
            pl.BlockSpec((1, 3 * d), lambda i: (0, 0)),
            pl.BlockSpec((1, 3 * d), lambda i: (0, 0)),
            pl.BlockSpec((d, d), lambda i: (0, 0)),
        ],
        out_specs=[
            pl.BlockSpec((_BLK, d), lambda i: (i, 0)),
            pl.BlockSpec((_BLK, d), lambda i: (i, 0)),
        ],
        out_shape=[
            jax.ShapeDtypeStruct((n_pad, d), jnp.float32),
            jax.ShapeDtypeStruct((n_pad, d), jnp.float32),
        ],
    )(parts, h, w_ih, w_hh, b_ih, b_hh, w_next)


def _pool_body(num_graphs, d, b_ref, h_ref, l1w_ref, l1b_ref, l2w_ref,
               l2b_ref, o_ref, sums_ref, counts_ref):
    i = pl.program_id(0)
    nblk = pl.num_programs(0)

    @pl.when(i == 0)
    def _():
        sums_ref[...] = jnp.zeros_like(sums_ref)
        counts_ref[...] = jnp.zeros_like(counts_ref)

    batch = b_ref[0]
    gids = lax.broadcasted_iota(jnp.int32, (num_graphs, 1), 0)
    onehot = (batch == gids).astype(jnp.float32)
    h = h_ref[...]
    sums_ref[...] += jnp.dot(onehot, h, preferred_element_type=jnp.float32)
    counts_ref[...] = counts_ref[...] + jnp.sum(onehot, axis=1, keepdims=True)

    @pl.when(i == nblk - 1)
    def _():
        gr = sums_ref[...] / jnp.maximum(counts_ref[...], 1.0)
        cdims = (((1,), (1,)), ((), ()))
        hid = lax.dot_general(gr, l1w_ref[...], cdims,
                              preferred_element_type=jnp.float32) + l1b_ref[...]
        hid = jnp.maximum(hid, 0.0)
        logits = lax.dot_general(hid, l2w_ref[...], cdims,
                                 preferred_element_type=jnp.float32) + l2b_ref[...]
        o_ref[...] = jax.nn.sigmoid(logits)


def _pool_mlp(h, batch_r3, l1w, l1b, l2w_pad, l2b_pad, num_graphs, n_pad, d,
              hid_d):
    grid = n_pad // _BLK
    return pl.pallas_call(
        functools.partial(_pool_body, num_graphs, d),
        grid=(grid,),
        in_specs=[
            pl.BlockSpec((1, 1, _BLK), lambda i: (i, 0, 0)),
            pl.BlockSpec((_BLK, d), lambda i: (i, 0)),
            pl.BlockSpec((hid_d, d), lambda i: (0, 0)),
            pl.BlockSpec((1, hid_d), lambda i: (0, 0)),
            pl.BlockSpec((hid_d, hid_d), lambda i: (0, 0)),
            pl.BlockSpec((1, hid_d), lambda i: (0, 0)),
        ],
        out_specs=pl.BlockSpec((num_graphs, hid_d), lambda i: (0, 0)),
        out_shape=jax.ShapeDtypeStruct((num_graphs, hid_d), jnp.float32),
        scratch_shapes=[
            pltpu.VMEM((num_graphs, d), jnp.float32),
            pltpu.VMEM((num_graphs, 1), jnp.float32),
        ],
    )(batch_r3, h, l1w, l1b, l2w_pad, l2b_pad)


def kernel(x, edge_index, batch, ggc_weight, w_ih, w_hh, b_ih, b_hh,
           lin1_w, lin1_b, lin2_w, lin2_b):
    n, d = x.shape
    num_layers = ggc_weight.shape[0]
    e = edge_index.shape[1]
    num_graphs = 256
    hid_d = lin1_w.shape[0]

    n_pad = ((n + _BLK) // _BLK) * _BLK
    chunks = (e + _NW * _K - 1) // (_NW * _K)
    chunks += chunks % 2
    e_pad = chunks * _NW * _K

    src = edge_index[0].astype(jnp.int32)
    dst = edge_index[1].astype(jnp.int32)
    src_r = jnp.concatenate(
        [src, jnp.full((e_pad - e,), n, jnp.int32)]).reshape(_NW, chunks * _K)
    dst_r = jnp.concatenate(
        [dst, jnp.zeros((e_pad - e,), jnp.int32)]).reshape(_NW, chunks, _K)

    zeros = jnp.zeros((n, d), jnp.float32)
    h = jnp.pad(x.astype(jnp.float32), ((0, n_pad - n), (0, 0)))

    batch_pad = jnp.concatenate(
        [batch.astype(jnp.int32),
         jnp.full((n_pad - n,), num_graphs, jnp.int32)]).reshape(
             n_pad // _BLK, 1, _BLK)

    b_ih2 = b_ih.astype(jnp.float32).reshape(1, 3 * d)
    b_hh2 = b_hh.astype(jnp.float32).reshape(1, 3 * d)
    l1b2 = lin1_b.astype(jnp.float32).reshape(1, hid_d)
    l2w_pad = jnp.zeros((hid_d, hid_d), jnp.float32).at[0].set(
        lin2_w[0].astype(jnp.float32))
    l2b_pad = jnp.zeros((1, hid_d), jnp.float32).at[0, 0].set(
        lin2_b[0].astype(jnp.float32))

    m = _mm(h, ggc_weight[0], n_pad, d)
    for i in range(num_layers):
        parts = _sc_agg(m, src_r, dst_r, zeros, n, n_pad, d, chunks)
        w_next = ggc_weight[(i + 1) % num_layers]
        h, m = _gru(parts, h, w_ih, w_hh, b_ih2, b_hh2, w_next,
                    n, n_pad, d)

    probs_pad = _pool_mlp(h, batch_pad, lin1_w, l1b2, l2w_pad, l2b_pad,
                          num_graphs, n_pad, d, hid_d)
    return probs_pad[:, :1]


# speedup vs baseline: 1.0246x; 1.0246x over previous
import functools

import jax
import jax.numpy as jnp
from jax import lax
from jax.experimental import pallas as pl
from jax.experimental.pallas import tpu as pltpu
from jax.experimental.pallas import tpu_sc as plsc

_NC = 2
_NS = 16
_NW = _NC * _NS
_K = 128
_BLK = 128


def _sc_agg(m, src_r, dst_r, zeros, n_agg, n_pad, d, chunks):
    mesh = plsc.VectorSubcoreMesh(core_axis_name="c", subcore_axis_name="s")

    @functools.partial(
        pl.kernel,
        mesh=mesh,
        out_type=jax.ShapeDtypeStruct((_NC, n_pad, d), jnp.float32),
        scratch_types=[
            pltpu.VMEM((chunks, _K), jnp.int32),
            pltpu.VMEM((chunks, _K), jnp.int32),
            pltpu.VMEM((_K, d), jnp.float32),
            pltpu.VMEM_SHARED((n_agg, d), jnp.float32),
            pltpu.SemaphoreType.DMA,
        ],
    )
    def body(m_hbm, src_hbm, dst_hbm, z_hbm, out_hbm, src_v, dst_v,
             rows_v, agg_sh, g0):
        c = lax.axis_index("c")
        s = lax.axis_index("s")
        wid = s * _NC + c
        zrows = n_agg // 10

        @pl.when(s < 10)
        def _():
            sl = pl.ds(s * zrows, zrows)
            pltpu.sync_copy(z_hbm.at[sl], agg_sh.at[sl])

        pltpu.sync_copy(src_hbm.at[wid], src_v)
        pltpu.sync_copy(dst_hbm.at[wid], dst_v)
        plsc.subcore_barrier()

        def step(j, carry):
            pltpu.async_copy(m_hbm.at[src_v.at[j]], rows_v, g0).wait()
            pltpu.sync_copy(rows_v, agg_sh.at[dst_v.at[j]], add=True)
            return carry

        lax.fori_loop(0, chunks, step, 0)
        plsc.subcore_barrier()

        @pl.when(s < 10)
        def _():
            sl = pl.ds(s * zrows, zrows)
            pltpu.sync_copy(agg_sh.at[sl], out_hbm.at[c].at[sl])

    return body(m, src_r, dst_r, zeros)


def _mm_body(h_ref, w_ref, o_ref):
    o_ref[...] = jnp.dot(h_ref[...], w_ref[...],
                         preferred_element_type=jnp.float32)


def _mm(h, w, n_pad, d):
    grid = n_pad // _BLK
    return pl.pallas_call(
        _mm_body,
        grid=(grid,),
        in_specs=[
            pl.BlockSpec((_BLK, d), lambda i: (i, 0)),
            pl.BlockSpec((d, d), lambda i: (0, 0)),
        ],
        out_specs=pl.BlockSpec((_BLK, d), lambda i: (i, 0)),
        out_shape=jax.ShapeDtypeStruct((n_pad, d), jnp.float32),
    )(h, w)


def _gru_body(n_nodes, d, parts_ref, h_ref, wih_ref, whh_ref, bih_ref,
              bhh_ref, wnext_ref, h_out_ref, m_out_ref):
    i = pl.program_id(0)
    agg = parts_ref[0] + parts_ref[1]
    h = h_ref[...]
    cdims = (((1,), (1,)), ((), ()))
    gi = lax.dot_general(agg, wih_ref[...], cdims,
                         preferred_element_type=jnp.float32) + bih_ref[...]
    gh = lax.dot_general(h, whh_ref[...], cdims,
                         preferred_element_type=jnp.float32) + bhh_ref[...]
    r = jax.nn.sigmoid(gi[:, :d] + gh[:, :d])
    z = jax.nn.sigmoid(gi[:, d:2 * d] + gh[:, d:2 * d])
    n = jnp.tanh(gi[:, 2 * d:] + r * gh[:, 2 * d:])
    hn = (1.0 - z) * n + z * h
    row = i * _BLK + lax.broadcasted_iota(jnp.int32, (_BLK, 1), 0)
    hn = jnp.where(row < n_nodes, hn, 0.0)
    h_out_ref[...] = hn
    m_out_ref[...] = jnp.dot(hn, wnext_ref[...],
                             preferred_element_type=jnp.float32)


def _gru(parts, h, w_ih, w_hh, b_ih, b_hh, w_next, n_nodes, n_pad, d):
    grid = n_pad // _BLK
    return pl.pallas_call(
        functools.partial(_gru_body, n_nodes, d),
        grid=(grid,),
        in_specs=[
            pl.BlockSpec((2, _BLK, d), lambda i: (0, i, 0)),
            pl.BlockSpec((_BLK, d), lambda i: (i, 0)),
            pl.BlockSpec((3 * d, d), lambda i: (0, 0)),
            pl.BlockSpec((3 * d, d), lambda i: (0, 0)),
            pl.BlockSpec((1, 3 * d), lambda i: (0, 0)),
            pl.BlockSpec((1, 3 * d), lambda i: (0, 0)),
            pl.BlockSpec((d, d), lambda i: (0, 0)),
        ],
        out_specs=[
            pl.BlockSpec((_BLK, d), lambda i: (i, 0)),
            pl.BlockSpec((_BLK, d), lambda i: (i, 0)),
        ],
        out_shape=[
            jax.ShapeDtypeStruct((n_pad, d), jnp.float32),
            jax.ShapeDtypeStruct((n_pad, d), jnp.float32),
        ],
    )(parts, h, w_ih, w_hh, b_ih, b_hh, w_next)


def _pool_body(num_graphs, d, b_ref, h_ref, l1w_ref, l1b_ref, l2w_ref,
               l2b_ref, o_ref, sums_ref, counts_ref):
    i = pl.program_id(0)
    nblk = pl.num_programs(0)

    @pl.when(i == 0)
    def _():
        sums_ref[...] = jnp.zeros_like(sums_ref)
        counts_ref[...] = jnp.zeros_like(counts_ref)

    batch = b_ref[0]
    gids = lax.broadcasted_iota(jnp.int32, (num_graphs, 1), 0)
    onehot = (batch == gids).astype(jnp.float32)
    h = h_ref[...]
    sums_ref[...] += jnp.dot(onehot, h, preferred_element_type=jnp.float32)
    counts_ref[...] = counts_ref[...] + jnp.sum(onehot, axis=1, keepdims=True)

    @pl.when(i == nblk - 1)
    def _():
        gr = sums_ref[...] / jnp.maximum(counts_ref[...], 1.0)
        cdims = (((1,), (1,)), ((), ()))
        hid = lax.dot_general(gr, l1w_ref[...], cdims,
                              preferred_element_type=jnp.float32) + l1b_ref[...]
        hid = jnp.maximum(hid, 0.0)
        logits = lax.dot_general(hid, l2w_ref[...], cdims,
                                 preferred_element_type=jnp.float32) + l2b_ref[...]
        o_ref[...] = jax.nn.sigmoid(logits)


def _pool_mlp(h, batch_r3, l1w, l1b, l2w_pad, l2b_pad, num_graphs, n_pad, d,
              hid_d):
    grid = n_pad // _BLK
    return pl.pallas_call(
        functools.partial(_pool_body, num_graphs, d),
        grid=(grid,),
        in_specs=[
            pl.BlockSpec((1, 1, _BLK), lambda i: (i, 0, 0)),
            pl.BlockSpec((_BLK, d), lambda i: (i, 0)),
            pl.BlockSpec((hid_d, d), lambda i: (0, 0)),
            pl.BlockSpec((1, hid_d), lambda i: (0, 0)),
            pl.BlockSpec((hid_d, hid_d), lambda i: (0, 0)),
            pl.BlockSpec((1, hid_d), lambda i: (0, 0)),
        ],
        out_specs=pl.BlockSpec((num_graphs, hid_d), lambda i: (0, 0)),
        out_shape=jax.ShapeDtypeStruct((num_graphs, hid_d), jnp.float32),
        scratch_shapes=[
            pltpu.VMEM((num_graphs, d), jnp.float32),
            pltpu.VMEM((num_graphs, 1), jnp.float32),
        ],
    )(batch_r3, h, l1w, l1b, l2w_pad, l2b_pad)


def kernel(x, edge_index, batch, ggc_weight, w_ih, w_hh, b_ih, b_hh,
           lin1_w, lin1_b, lin2_w, lin2_b):
    n, d = x.shape
    num_layers = ggc_weight.shape[0]
    e = edge_index.shape[1]
    num_graphs = 256
    hid_d = lin1_w.shape[0]

    n_pad = ((n + _BLK) // _BLK) * _BLK
    chunks = (e + _NW * _K - 1) // (_NW * _K)
    chunks += chunks % 2
    e_pad = chunks * _NW * _K

    src = edge_index[0].astype(jnp.int32)
    dst = edge_index[1].astype(jnp.int32)
    src_r = jnp.concatenate(
        [src, jnp.full((e_pad - e,), n, jnp.int32)]).reshape(_NW, chunks, _K)
    dst_r = jnp.concatenate(
        [dst, jnp.zeros((e_pad - e,), jnp.int32)]).reshape(_NW, chunks, _K)

    zeros = jnp.zeros((n, d), jnp.float32)
    h = jnp.pad(x.astype(jnp.float32), ((0, n_pad - n), (0, 0)))

    batch_pad = jnp.concatenate(
        [batch.astype(jnp.int32),
         jnp.full((n_pad - n,), num_graphs, jnp.int32)]).reshape(
             n_pad // _BLK, 1, _BLK)

    b_ih2 = b_ih.astype(jnp.float32).reshape(1, 3 * d)
    b_hh2 = b_hh.astype(jnp.float32).reshape(1, 3 * d)
    l1b2 = lin1_b.astype(jnp.float32).reshape(1, hid_d)
    l2w_pad = jnp.zeros((hid_d, hid_d), jnp.float32).at[0].set(
        lin2_w[0].astype(jnp.float32))
    l2b_pad = jnp.zeros((1, hid_d), jnp.float32).at[0, 0].set(
        lin2_b[0].astype(jnp.float32))

    m = _mm(h, ggc_weight[0], n_pad, d)
    for i in range(num_layers):
        parts = _sc_agg(m, src_r, dst_r, zeros, n, n_pad, d, chunks)
        w_next = ggc_weight[(i + 1) % num_layers]
        h, m = _gru(parts, h, w_ih, w_hh, b_ih2, b_hh2, w_next,
                    n, n_pad, d)

    probs_pad = _pool_mlp(h, batch_pad, lin1_w, l1b2, l2w_pad, l2b_pad,
                          num_graphs, n_pad, d, hid_d)
    return probs_pad[:, :1]


# speedup vs baseline: 1.4171x; 1.3831x over previous
import functools

import jax
import jax.numpy as jnp
from jax import lax
from jax.experimental import pallas as pl
from jax.experimental.pallas import tpu as pltpu
from jax.experimental.pallas import tpu_sc as plsc

_NC = 2
_NS = 16
_NW = _NC * _NS
_K = 128
_BLK = 128


def _sc_agg(m, src_r, dst_r, zeros, n_agg, n_pad, d, chunks):
    mesh = plsc.VectorSubcoreMesh(core_axis_name="c", subcore_axis_name="s")

    @functools.partial(
        pl.kernel,
        mesh=mesh,
        out_type=jax.ShapeDtypeStruct((_NC, n_pad, d), jnp.float32),
        scratch_types=[
            pltpu.VMEM((chunks, _K), jnp.int32),
            pltpu.VMEM((chunks, _K), jnp.int32),
            pltpu.VMEM((_K, d), jnp.float32),
            pltpu.VMEM_SHARED((n_agg, d), jnp.float32),
            pltpu.SemaphoreType.DMA,
        ],
    )
    def body(m_hbm, src_hbm, dst_hbm, z_hbm, out_hbm, src_v, dst_v,
             rows_v, agg_sh, g0):
        c = lax.axis_index("c")
        s = lax.axis_index("s")
        wid = s * _NC + c
        zrows = n_agg // 10

        @pl.when(s < 10)
        def _():
            sl = pl.ds(s * zrows, zrows)
            pltpu.sync_copy(z_hbm.at[sl], agg_sh.at[sl])

        pltpu.sync_copy(src_hbm.at[wid], src_v)
        pltpu.sync_copy(dst_hbm.at[wid], dst_v)
        plsc.subcore_barrier()

        def step(j, carry):
            pltpu.async_copy(m_hbm.at[src_v.at[j]], rows_v, g0).wait()
            pltpu.sync_copy(rows_v, agg_sh.at[dst_v.at[j]], add=True)
            return carry

        lax.fori_loop(0, chunks, step, 0)
        plsc.subcore_barrier()

        @pl.when(s < 10)
        def _():
            sl = pl.ds(s * zrows, zrows)
            pltpu.sync_copy(agg_sh.at[sl], out_hbm.at[c].at[sl])

    return body(m, src_r, dst_r, zeros)


def _mm_body(h_ref, w_ref, o_ref):
    o_ref[...] = jnp.dot(h_ref[...], w_ref[...],
                         preferred_element_type=jnp.float32)


def _mm(h, w, n_pad, d):
    grid = n_pad // _BLK
    return pl.pallas_call(
        _mm_body,
        grid=(grid,),
        in_specs=[
            pl.BlockSpec((_BLK, d), lambda i: (i, 0)),
            pl.BlockSpec((d, d), lambda i: (0, 0)),
        ],
        out_specs=pl.BlockSpec((_BLK, d), lambda i: (i, 0)),
        out_shape=jax.ShapeDtypeStruct((n_pad, d), jnp.float32),
    )(h, w)


def _gru_body(n_nodes, d, parts_ref, h_ref, wih_ref, whh_ref, bih_ref,
              bhh_ref, wnext_ref, h_out_ref, m_out_ref):
    i = pl.program_id(0)
    agg = parts_ref[0] + parts_ref[1]
    h = h_ref[...]
    cdims = (((1,), (1,)), ((), ()))
    gi = lax.dot_general(agg, wih_ref[...], cdims,
                         preferred_element_type=jnp.float32) + bih_ref[...]
    gh = lax.dot_general(h, whh_ref[...], cdims,
                         preferred_element_type=jnp.float32) + bhh_ref[...]
    r = jax.nn.sigmoid(gi[:, :d] + gh[:, :d])
    z = jax.nn.sigmoid(gi[:, d:2 * d] + gh[:, d:2 * d])
    n = jnp.tanh(gi[:, 2 * d:] + r * gh[:, 2 * d:])
    hn = (1.0 - z) * n + z * h
    row = i * _BLK + lax.broadcasted_iota(jnp.int32, (_BLK, 1), 0)
    hn = jnp.where(row < n_nodes, hn, 0.0)
    h_out_ref[...] = hn
    m_out_ref[...] = jnp.dot(hn, wnext_ref[...],
                             preferred_element_type=jnp.float32)


def _gru(parts, h, w_ih, w_hh, b_ih, b_hh, w_next, n_nodes, n_pad, d):
    grid = n_pad // _BLK
    return pl.pallas_call(
        functools.partial(_gru_body, n_nodes, d),
        grid=(grid,),
        in_specs=[
            pl.BlockSpec((2, _BLK, d), lambda i: (0, i, 0)),
            pl.BlockSpec((_BLK, d), lambda i: (i, 0)),
            pl.BlockSpec((3 * d, d), lambda i: (0, 0)),
            pl.BlockSpec((3 * d, d), lambda i: (0, 0)),
            pl.BlockSpec((1, 3 * d), lambda i: (0, 0)),
            pl.BlockSpec((1, 3 * d), lambda i: (0, 0)),
            pl.BlockSpec((d, d), lambda i: (0, 0)),
        ],
        out_specs=[
            pl.BlockSpec((_BLK, d), lambda i: (i, 0)),
            pl.BlockSpec((_BLK, d), lambda i: (i, 0)),
        ],
        out_shape=[
            jax.ShapeDtypeStruct((n_pad, d), jnp.float32),
            jax.ShapeDtypeStruct((n_pad, d), jnp.float32),
        ],
    )(parts, h, w_ih, w_hh, b_ih, b_hh, w_next)


def _pool_body(num_graphs, d, b_ref, h_ref, l1w_ref, l1b_ref, l2w_ref,
               l2b_ref, o_ref, sums_ref, counts_ref):
    i = pl.program_id(0)
    nblk = pl.num_programs(0)

    @pl.when(i == 0)
    def _():
        sums_ref[...] = jnp.zeros_like(sums_ref)
        counts_ref[...] = jnp.zeros_like(counts_ref)

    batch = b_ref[0]
    gids = lax.broadcasted_iota(jnp.int32, (num_graphs, 1), 0)
    onehot = (batch == gids).astype(jnp.float32)
    h = h_ref[...]
    sums_ref[...] += jnp.dot(onehot, h, preferred_element_type=jnp.float32)
    counts_ref[...] = counts_ref[...] + jnp.sum(onehot, axis=1, keepdims=True)

    @pl.when(i == nblk - 1)
    def _():
        gr = sums_ref[...] / jnp.maximum(counts_ref[...], 1.0)
        cdims = (((1,), (1,)), ((), ()))
        hid = lax.dot_general(gr, l1w_ref[...], cdims,
                              preferred_element_type=jnp.float32) + l1b_ref[...]
        hid = jnp.maximum(hid, 0.0)
        logits = lax.dot_general(hid, l2w_ref[...], cdims,
                                 preferred_element_type=jnp.float32) + l2b_ref[...]
        o_ref[...] = jax.nn.sigmoid(logits)


def _pool_mlp(h, batch_r3, l1w, l1b, l2w_pad, l2b_pad, num_graphs, n_pad, d,
              hid_d):
    grid = n_pad // _BLK
    return pl.pallas_call(
        functools.partial(_pool_body, num_graphs, d),
        grid=(grid,),
        in_specs=[
            pl.BlockSpec((1, 1, _BLK), lambda i: (i, 0, 0)),
            pl.BlockSpec((_BLK, d), lambda i: (i, 0)),
            pl.BlockSpec((hid_d, d), lambda i: (0, 0)),
            pl.BlockSpec((1, hid_d), lambda i: (0, 0)),
            pl.BlockSpec((hid_d, hid_d), lambda i: (0, 0)),
            pl.BlockSpec((1, hid_d), lambda i: (0, 0)),
        ],
        out_specs=pl.BlockSpec((num_graphs, hid_d), lambda i: (0, 0)),
        out_shape=jax.ShapeDtypeStruct((num_graphs, hid_d), jnp.float32),
        scratch_shapes=[
            pltpu.VMEM((num_graphs, d), jnp.float32),
            pltpu.VMEM((num_graphs, 1), jnp.float32),
        ],
    )(batch_r3, h, l1w, l1b, l2w_pad, l2b_pad)


def kernel(x, edge_index, batch, ggc_weight, w_ih, w_hh, b_ih, b_hh,
           lin1_w, lin1_b, lin2_w, lin2_b):
    n, d = x.shape
    num_layers = ggc_weight.shape[0]
    e = edge_index.shape[1]
    num_graphs = 256
    hid_d = lin1_w.shape[0]

    n_pad = ((n + _BLK) // _BLK) * _BLK
    chunks = (e + _NW * _K - 1) // (_NW * _K)
    e_pad = chunks * _NW * _K

    src = edge_index[0].astype(jnp.int32)
    dst = edge_index[1].astype(jnp.int32)
    src_r = jnp.concatenate(
        [src, jnp.full((e_pad - e,), n, jnp.int32)]).reshape(_NW, chunks, _K)
    dst_r = jnp.concatenate(
        [dst, jnp.arange(e_pad - e, dtype=jnp.int32) % n]).reshape(
            _NW, chunks, _K)

    zeros = jnp.zeros((n, d), jnp.float32)
    h = jnp.pad(x.astype(jnp.float32), ((0, n_pad - n), (0, 0)))

    batch_pad = jnp.concatenate(
        [batch.astype(jnp.int32),
         jnp.full((n_pad - n,), num_graphs, jnp.int32)]).reshape(
             n_pad // _BLK, 1, _BLK)

    b_ih2 = b_ih.astype(jnp.float32).reshape(1, 3 * d)
    b_hh2 = b_hh.astype(jnp.float32).reshape(1, 3 * d)
    l1b2 = lin1_b.astype(jnp.float32).reshape(1, hid_d)
    l2w_pad = jnp.zeros((hid_d, hid_d), jnp.float32).at[0].set(
        lin2_w[0].astype(jnp.float32))
    l2b_pad = jnp.zeros((1, hid_d), jnp.float32).at[0, 0].set(
        lin2_b[0].astype(jnp.float32))

    m = _mm(h, ggc_weight[0], n_pad, d)
    for i in range(num_layers):
        parts = _sc_agg(m, src_r, dst_r, zeros, n, n_pad, d, chunks)
        w_next = ggc_weight[(i + 1) % num_layers]
        h, m = _gru(parts, h, w_ih, w_hh, b_ih2, b_hh2, w_next,
                    n, n_pad, d)

    probs_pad = _pool_mlp(h, batch_pad, lin1_w, l1b2, l2w_pad, l2b_pad,
                          num_graphs, n_pad, d, hid_d)
    return probs_pad[:, :1]
